# Initial kernel scaffold; baseline (speedup 1.0000x reference)
import jax
import jax.numpy as jnp
from jax.experimental import pallas as pl


def kernel(input, W):
    raise NotImplementedError("write your pallas kernel here")


# speedup vs baseline: 4.5466x; 4.5466x over previous
import functools

import jax
import jax.numpy as jnp
from jax import lax
from jax.experimental import pallas as pl
from jax.experimental.pallas import tpu as pltpu
from jax.experimental.pallas import tpu_sc as plsc

NINP = 64
NUM_CORES = 2
NUM_SUBCORES = 16
NW = NUM_CORES * NUM_SUBCORES


def _gather_call(n_rows, chunk, n_chunks, table, idx):
    mesh = plsc.VectorSubcoreMesh(
        core_axis_name="c", subcore_axis_name="s",
        num_cores=NUM_CORES, num_subcores=NUM_SUBCORES,
    )
    b_per_w = n_rows // NW

    @functools.partial(
        pl.kernel,
        out_type=jax.ShapeDtypeStruct((n_rows, NINP), jnp.float32),
        mesh=mesh,
        compiler_params=pltpu.CompilerParams(use_tc_tiling_on_sc=False),
        scratch_types=[
            pltpu.VMEM((chunk,), jnp.int32),
            pltpu.VMEM((chunk, NINP), jnp.float32),
            pltpu.SemaphoreType.DMA,
        ],
    )
    def gather_kernel(idx_hbm, table_hbm, out_hbm, idx_v, rows_v, sem):
        wid = lax.axis_index("s") * NUM_CORES + lax.axis_index("c")
        base = wid * b_per_w

        def body(i, _):
            start = base + i * chunk
            pltpu.sync_copy(idx_hbm.at[pl.ds(start, chunk)], idx_v)
            pltpu.async_copy(table_hbm.at[idx_v], rows_v, sem).wait()
            pltpu.sync_copy(rows_v, out_hbm.at[pl.ds(start, chunk)])
            return 0

        lax.fori_loop(0, n_chunks, body, 0)

    return gather_kernel(idx, table)


def kernel(input, W):
    batch, hist = input.shape
    n_rows = batch * hist
    idx = input.reshape(n_rows).astype(jnp.int32)
    chunk = 800
    n_chunks = (n_rows // NW) // chunk
    out = _gather_call(n_rows, chunk, n_chunks, W, idx)
    return out.reshape(batch, hist, NINP)


# speedup vs baseline: 4.6847x; 1.0304x over previous
import functools

import jax
import jax.numpy as jnp
from jax import lax
from jax.experimental import pallas as pl
from jax.experimental.pallas import tpu as pltpu
from jax.experimental.pallas import tpu_sc as plsc

NINP = 64
NUM_CORES = 2
NUM_SUBCORES = 16
NW = NUM_CORES * NUM_SUBCORES


def _gather_call(n_rows, chunk, n_chunks, table, idx):
    mesh = plsc.VectorSubcoreMesh(
        core_axis_name="c", subcore_axis_name="s",
        num_cores=NUM_CORES, num_subcores=NUM_SUBCORES,
    )
    b_per_w = n_rows // NW

    @functools.partial(
        pl.kernel,
        out_type=jax.ShapeDtypeStruct((n_rows, NINP), jnp.float32),
        mesh=mesh,
        compiler_params=pltpu.CompilerParams(use_tc_tiling_on_sc=False),
        scratch_types=[
            pltpu.VMEM((b_per_w,), jnp.int32),
            pltpu.VMEM((2, chunk, NINP), jnp.float32),
            pltpu.SemaphoreType.DMA,
            pltpu.SemaphoreType.DMA,
        ],
    )
    def gather_kernel(idx_hbm, table_hbm, out_hbm, idx_v, rows_v, sem_g, sem_w):
        wid = lax.axis_index("s") * NUM_CORES + lax.axis_index("c")
        base = wid * b_per_w

        pltpu.sync_copy(idx_hbm.at[pl.ds(base, b_per_w)], idx_v)

        def gather_start(i):
            return pltpu.async_copy(
                table_hbm.at[idx_v.at[pl.ds(i * chunk, chunk)]],
                rows_v.at[i % 2], sem_g)

        def wb_start(i):
            return pltpu.async_copy(
                rows_v.at[i % 2],
                out_hbm.at[pl.ds(base + i * chunk, chunk)], sem_w)

        copies = [gather_start(0)]
        wbs = []
        for i in range(n_chunks):
            if i + 1 < n_chunks:
                if i >= 1:
                    wbs[i - 1].wait()
                copies.append(gather_start(i + 1))
            copies[i].wait()
            wbs.append(wb_start(i))
        wbs[n_chunks - 2].wait()
        wbs[n_chunks - 1].wait()

    return gather_kernel(idx, table)


def kernel(input, W):
    batch, hist = input.shape
    n_rows = batch * hist
    idx = input.reshape(n_rows).astype(jnp.int32)
    chunk = 800
    n_chunks = (n_rows // NW) // chunk
    out = _gather_call(n_rows, chunk, n_chunks, W, idx)
    return out.reshape(batch, hist, NINP)
